# baseline (speedup 1.0000x reference)
import functools

import jax
import jax.numpy as jnp
from jax.experimental import pallas as pl

_B, _P, _K = 16, 1024, 20


def _edgeconv_kernel(x_ref, wd_ref, wb_ref, b1_ref, w2_ref, b2_ref,
                     w3_ref, b3_ref, out_ref, *, k, mdt):
    x = x_ref[0]
    p = x.shape[0]
    f32 = jnp.float32

    dots = jax.lax.dot_general(x, x, (((1,), (1,)), ((), ())),
                               preferred_element_type=f32)
    xx = x * x
    sq_row = jnp.sum(xx, axis=1, keepdims=True)
    ones = jnp.ones((1, x.shape[1]), f32)
    sq_col = jax.lax.dot_general(ones, xx, (((1,), (1,)), ((), ())),
                                 preferred_element_type=f32)
    d_cur = sq_row + sq_col - 2.0 * dots

    u = jnp.dot(x, wd_ref[...], preferred_element_type=f32) + b1_ref[...]
    v = jnp.dot(x, wb_ref[...], preferred_element_type=f32)
    if mdt == jnp.float32:
        v_hi = v.astype(jnp.bfloat16)
        v_lo = (v - v_hi.astype(f32)).astype(jnp.bfloat16)
    else:
        v_hi = v.astype(mdt)
        v_lo = None

    cols = jax.lax.broadcasted_iota(jnp.int32, (p, p), 1)
    w2 = w2_ref[...].astype(mdt)
    b2 = b2_ref[...]
    w3 = w3_ref[...].astype(mdt)
    b3 = b3_ref[...]

    acc = jnp.zeros((p, w3.shape[1]), f32)
    for _ in range(k):
        idx = jnp.argmin(d_cur, axis=1).astype(jnp.int32)[:, None]
        hit = cols == idx
        onehot = hit.astype(jnp.bfloat16)
        xj = jnp.dot(onehot, v_hi, preferred_element_type=f32)
        if v_lo is not None:
            xj = xj + jnp.dot(onehot, v_lo, preferred_element_type=f32)
        h = jax.nn.relu(u + xj)
        h = jax.nn.relu(jnp.dot(h.astype(mdt), w2,
                                preferred_element_type=f32) + b2)
        h = jax.nn.relu(jnp.dot(h.astype(mdt), w3,
                                preferred_element_type=f32) + b3)
        acc = jnp.maximum(acc, h)
        d_cur = jnp.where(hit, jnp.inf, d_cur)
    out_ref[0] = acc


def _head_kernel(x_ref, w0_ref, b0_ref, w1_ref, b1_ref, w2_ref, b2_ref,
                 w3_ref, b3_ref, out_ref):
    f32 = jnp.float32
    bf16 = jnp.bfloat16
    x = x_ref[0].astype(bf16)
    h = jax.nn.relu(jnp.dot(x, w0_ref[...].astype(bf16),
                            preferred_element_type=f32)
                    + b0_ref[...])
    g = jnp.max(h, axis=0, keepdims=True)
    g = jax.nn.relu(jnp.dot(g, w1_ref[...], preferred_element_type=f32)
                    + b1_ref[...])
    g = jax.nn.relu(jnp.dot(g, w2_ref[...], preferred_element_type=f32)
                    + b2_ref[...])
    logits = jnp.dot(g, w3_ref[...], preferred_element_type=f32) + b3_ref[...]
    m = jnp.max(logits, axis=1, keepdims=True)
    z = logits - m
    lse = jnp.log(jnp.sum(jnp.exp(z), axis=1, keepdims=True))
    out_ref[0] = z - lse


def _full_spec(shape):
    return pl.BlockSpec(shape, lambda *a: (0,) * len(shape))


def _edgeconv(xb, layers, din, mdt):
    w1 = layers[0]["w"]
    wd = w1[:din] - w1[din:]
    wb = w1[din:]
    b1 = layers[0]["b"].reshape(1, -1)
    w2, b2 = layers[1]["w"], layers[1]["b"].reshape(1, -1)
    w3, b3 = layers[2]["w"], layers[2]["b"].reshape(1, -1)
    fout = w3.shape[1]
    fn = functools.partial(_edgeconv_kernel, k=_K, mdt=mdt)
    return pl.pallas_call(
        fn,
        grid=(_B,),
        in_specs=[
            pl.BlockSpec((1, _P, din), lambda b: (b, 0, 0)),
            _full_spec(wd.shape), _full_spec(wb.shape),
            _full_spec(b1.shape),
            _full_spec(w2.shape), _full_spec(b2.shape),
            _full_spec(w3.shape), _full_spec(b3.shape),
        ],
        out_specs=pl.BlockSpec((1, _P, fout), lambda b: (b, 0, 0)),
        out_shape=jax.ShapeDtypeStruct((_B, _P, fout), jnp.float32),
    )(xb, wd, wb, b1, w2, b2, w3, b3)


def _head(x2, params):
    w0, b0 = params["lin0"]["w"], params["lin0"]["b"].reshape(1, -1)
    w1, b1 = params["lin1"]["w"], params["lin1"]["b"].reshape(1, -1)
    w2, b2 = params["lin2"]["w"], params["lin2"]["b"].reshape(1, -1)
    w3, b3 = params["lin3"]["w"], params["lin3"]["b"].reshape(1, -1)
    nc = w3.shape[1]
    return pl.pallas_call(
        _head_kernel,
        grid=(_B,),
        in_specs=[
            pl.BlockSpec((1, _P, 256), lambda b: (b, 0, 0)),
            _full_spec(w0.shape), _full_spec(b0.shape),
            _full_spec(w1.shape), _full_spec(b1.shape),
            _full_spec(w2.shape), _full_spec(b2.shape),
            _full_spec(w3.shape), _full_spec(b3.shape),
        ],
        out_specs=pl.BlockSpec((1, 1, nc), lambda b: (b, 0, 0)),
        out_shape=jax.ShapeDtypeStruct((_B, 1, nc), jnp.float32),
    )(x2, w0, b0, w1, b1, w2, b2, w3, b3).reshape(_B, nc)


def kernel(pos, batch, params):
    del batch
    xb = pos.reshape(_B, _P, 3)
    x1 = _edgeconv(xb, params["c1"], 3, jnp.float32)
    x2 = _edgeconv(x1, params["c2"], 64, jnp.bfloat16)
    return _head(x2, params)


# speedup vs baseline: 1.3782x; 1.3782x over previous
import functools

import jax
import jax.numpy as jnp
from jax.experimental import pallas as pl

_B, _P, _K = 16, 1024, 20


def _edgeconv_kernel(x_ref, wd_ref, wb_ref, b1_ref, w2_ref, b2_ref,
                     w3_ref, b3_ref, out_ref, *, k, mdt):
    x = x_ref[0]
    p = x.shape[0]
    f32 = jnp.float32

    dots = jax.lax.dot_general(x, x, (((1,), (1,)), ((), ())),
                               preferred_element_type=f32)
    xx = x * x
    sq_row = jnp.sum(xx, axis=1, keepdims=True)
    ones = jnp.ones((1, x.shape[1]), f32)
    sq_col = jax.lax.dot_general(ones, xx, (((1,), (1,)), ((), ())),
                                 preferred_element_type=f32)
    d_cur = sq_row + sq_col - 2.0 * dots

    u = jnp.dot(x, wd_ref[...], preferred_element_type=f32) + b1_ref[...]
    v = jnp.dot(x, wb_ref[...], preferred_element_type=f32).astype(mdt)

    cols = jax.lax.broadcasted_iota(jnp.int32, (p, p), 1)
    w2 = w2_ref[...].astype(mdt)
    b2 = b2_ref[...]
    w3 = w3_ref[...].astype(mdt)
    b3 = b3_ref[...]

    acc = jnp.zeros((p, w3.shape[1]), f32)
    for _ in range(k):
        idx = jnp.argmin(d_cur, axis=1).astype(jnp.int32)[:, None]
        hit = cols == idx
        onehot = hit.astype(mdt)
        h = jax.nn.relu(u + jnp.dot(onehot, v, preferred_element_type=f32))
        h = jax.nn.relu(jnp.dot(h.astype(mdt), w2,
                                preferred_element_type=f32) + b2)
        h = jax.nn.relu(jnp.dot(h.astype(mdt), w3,
                                preferred_element_type=f32) + b3)
        acc = jnp.maximum(acc, h)
        d_cur = jnp.where(hit, jnp.inf, d_cur)
    out_ref[0] = acc


def _head_kernel(x_ref, w0_ref, b0_ref, w1_ref, b1_ref, w2_ref, b2_ref,
                 w3_ref, b3_ref, out_ref):
    f32 = jnp.float32
    bf16 = jnp.bfloat16
    x = x_ref[0].astype(bf16)
    h = jax.nn.relu(jnp.dot(x, w0_ref[...].astype(bf16),
                            preferred_element_type=f32)
                    + b0_ref[...])
    g = jnp.max(h, axis=0, keepdims=True)
    g = jax.nn.relu(jnp.dot(g, w1_ref[...], preferred_element_type=f32)
                    + b1_ref[...])
    g = jax.nn.relu(jnp.dot(g, w2_ref[...], preferred_element_type=f32)
                    + b2_ref[...])
    logits = jnp.dot(g, w3_ref[...], preferred_element_type=f32) + b3_ref[...]
    m = jnp.max(logits, axis=1, keepdims=True)
    z = logits - m
    lse = jnp.log(jnp.sum(jnp.exp(z), axis=1, keepdims=True))
    out_ref[0] = z - lse


def _full_spec(shape):
    return pl.BlockSpec(shape, lambda *a: (0,) * len(shape))


def _edgeconv(xb, layers, din, mdt):
    w1 = layers[0]["w"]
    wd = w1[:din] - w1[din:]
    wb = w1[din:]
    b1 = layers[0]["b"].reshape(1, -1)
    w2, b2 = layers[1]["w"], layers[1]["b"].reshape(1, -1)
    w3, b3 = layers[2]["w"], layers[2]["b"].reshape(1, -1)
    fout = w3.shape[1]
    fn = functools.partial(_edgeconv_kernel, k=_K, mdt=mdt)
    return pl.pallas_call(
        fn,
        grid=(_B,),
        in_specs=[
            pl.BlockSpec((1, _P, din), lambda b: (b, 0, 0)),
            _full_spec(wd.shape), _full_spec(wb.shape),
            _full_spec(b1.shape),
            _full_spec(w2.shape), _full_spec(b2.shape),
            _full_spec(w3.shape), _full_spec(b3.shape),
        ],
        out_specs=pl.BlockSpec((1, _P, fout), lambda b: (b, 0, 0)),
        out_shape=jax.ShapeDtypeStruct((_B, _P, fout), jnp.float32),
    )(xb, wd, wb, b1, w2, b2, w3, b3)


def _head(x2, params):
    w0, b0 = params["lin0"]["w"], params["lin0"]["b"].reshape(1, -1)
    w1, b1 = params["lin1"]["w"], params["lin1"]["b"].reshape(1, -1)
    w2, b2 = params["lin2"]["w"], params["lin2"]["b"].reshape(1, -1)
    w3, b3 = params["lin3"]["w"], params["lin3"]["b"].reshape(1, -1)
    nc = w3.shape[1]
    return pl.pallas_call(
        _head_kernel,
        grid=(_B,),
        in_specs=[
            pl.BlockSpec((1, _P, 256), lambda b: (b, 0, 0)),
            _full_spec(w0.shape), _full_spec(b0.shape),
            _full_spec(w1.shape), _full_spec(b1.shape),
            _full_spec(w2.shape), _full_spec(b2.shape),
            _full_spec(w3.shape), _full_spec(b3.shape),
        ],
        out_specs=pl.BlockSpec((1, 1, nc), lambda b: (b, 0, 0)),
        out_shape=jax.ShapeDtypeStruct((_B, 1, nc), jnp.float32),
    )(x2, w0, b0, w1, b1, w2, b2, w3, b3).reshape(_B, nc)


def kernel(pos, batch, params):
    del batch
    xb = pos.reshape(_B, _P, 3)
    x1 = _edgeconv(xb, params["c1"], 3, jnp.float32)
    x2 = _edgeconv(x1, params["c2"], 64, jnp.bfloat16)
    return _head(x2, params)


# speedup vs baseline: 1.3960x; 1.0129x over previous
import functools

import jax
import jax.numpy as jnp
from jax.experimental import pallas as pl

_B, _P, _K = 16, 1024, 20


def _conv_body(x, wd_ref, wb_ref, b1_ref, w2_ref, b2_ref, w3_ref, b3_ref,
               k, mdt):
    p = x.shape[0]
    f32 = jnp.float32

    dots = jax.lax.dot_general(x, x, (((1,), (1,)), ((), ())),
                               preferred_element_type=f32)
    xx = x * x
    sq_row = jnp.sum(xx, axis=1, keepdims=True)
    ones = jnp.ones((1, x.shape[1]), f32)
    sq_col = jax.lax.dot_general(ones, xx, (((1,), (1,)), ((), ())),
                                 preferred_element_type=f32)
    d_cur = sq_row + sq_col - 2.0 * dots

    u = jnp.dot(x, wd_ref[...], preferred_element_type=f32) + b1_ref[...]
    v = jnp.dot(x, wb_ref[...], preferred_element_type=f32).astype(mdt)

    cols = jax.lax.broadcasted_iota(jnp.int32, (p, p), 1)
    w2 = w2_ref[...].astype(mdt)
    b2 = b2_ref[...]
    w3 = w3_ref[...].astype(mdt)
    b3 = b3_ref[...]

    acc = jnp.zeros((p, w3.shape[1]), f32)
    for _ in range(k):
        idx = jnp.argmin(d_cur, axis=1).astype(jnp.int32)[:, None]
        hit = cols == idx
        onehot = hit.astype(mdt)
        h = jax.nn.relu(u + jnp.dot(onehot, v, preferred_element_type=f32))
        h = jax.nn.relu(jnp.dot(h.astype(mdt), w2,
                                preferred_element_type=f32) + b2)
        h = jax.nn.relu(jnp.dot(h.astype(mdt), w3,
                                preferred_element_type=f32) + b3)
        acc = jnp.maximum(acc, h)
        d_cur = jnp.where(hit, jnp.inf, d_cur)
    return acc


def _edgeconv1_kernel(x_ref, wd_ref, wb_ref, b1_ref, w2_ref, b2_ref,
                      w3_ref, b3_ref, out_ref, *, k, mdt):
    out_ref[0] = _conv_body(x_ref[0], wd_ref, wb_ref, b1_ref, w2_ref,
                            b2_ref, w3_ref, b3_ref, k, mdt)


def _edgeconv2_kernel(x_ref, wd_ref, wb_ref, b1_ref, w2_ref, b2_ref,
                      w3_ref, b3_ref, w0_ref, b0_ref, out_ref, *, k, mdt):
    f32 = jnp.float32
    bf16 = jnp.bfloat16
    x2 = _conv_body(x_ref[0], wd_ref, wb_ref, b1_ref, w2_ref, b2_ref,
                    w3_ref, b3_ref, k, mdt)
    h = jax.nn.relu(jnp.dot(x2.astype(bf16), w0_ref[...].astype(bf16),
                            preferred_element_type=f32)
                    + b0_ref[...])
    out_ref[0] = jnp.max(h, axis=0, keepdims=True)


def _tail_kernel(g_ref, w1_ref, b1_ref, w2_ref, b2_ref, w3_ref, b3_ref,
                 out_ref):
    f32 = jnp.float32
    g = jax.nn.relu(jnp.dot(g_ref[...], w1_ref[...],
                            preferred_element_type=f32) + b1_ref[...])
    g = jax.nn.relu(jnp.dot(g, w2_ref[...],
                            preferred_element_type=f32) + b2_ref[...])
    logits = jnp.dot(g, w3_ref[...], preferred_element_type=f32) + b3_ref[...]
    m = jnp.max(logits, axis=1, keepdims=True)
    z = logits - m
    lse = jnp.log(jnp.sum(jnp.exp(z), axis=1, keepdims=True))
    out_ref[...] = z - lse


def _full_spec(shape):
    return pl.BlockSpec(shape, lambda *a: (0,) * len(shape))


def _conv_args(layers, din):
    w1 = layers[0]["w"]
    return (w1[:din] - w1[din:], w1[din:], layers[0]["b"].reshape(1, -1),
            layers[1]["w"], layers[1]["b"].reshape(1, -1),
            layers[2]["w"], layers[2]["b"].reshape(1, -1))


def kernel(pos, batch, params):
    del batch
    xb = pos.reshape(_B, _P, 3)

    c1 = _conv_args(params["c1"], 3)
    x1 = pl.pallas_call(
        functools.partial(_edgeconv1_kernel, k=_K, mdt=jnp.float32),
        grid=(_B,),
        in_specs=[pl.BlockSpec((1, _P, 3), lambda b: (b, 0, 0))]
        + [_full_spec(a.shape) for a in c1],
        out_specs=pl.BlockSpec((1, _P, 64), lambda b: (b, 0, 0)),
        out_shape=jax.ShapeDtypeStruct((_B, _P, 64), jnp.float32),
    )(xb, *c1)

    c2 = _conv_args(params["c2"], 64)
    w0, b0 = params["lin0"]["w"], params["lin0"]["b"].reshape(1, -1)
    g = pl.pallas_call(
        functools.partial(_edgeconv2_kernel, k=_K, mdt=jnp.bfloat16),
        grid=(_B,),
        in_specs=[pl.BlockSpec((1, _P, 64), lambda b: (b, 0, 0))]
        + [_full_spec(a.shape) for a in c2 + (w0, b0)],
        out_specs=pl.BlockSpec((1, 1, 512), lambda b: (b, 0, 0)),
        out_shape=jax.ShapeDtypeStruct((_B, 1, 512), jnp.float32),
    )(x1, *c2, w0, b0).reshape(_B, 512)

    w1, b1 = params["lin1"]["w"], params["lin1"]["b"].reshape(1, -1)
    w2, b2 = params["lin2"]["w"], params["lin2"]["b"].reshape(1, -1)
    w3, b3 = params["lin3"]["w"], params["lin3"]["b"].reshape(1, -1)
    nc = w3.shape[1]
    targs = (g, w1, b1, w2, b2, w3, b3)
    return pl.pallas_call(
        _tail_kernel,
        in_specs=[_full_spec(a.shape) for a in targs],
        out_specs=_full_spec((_B, nc)),
        out_shape=jax.ShapeDtypeStruct((_B, nc), jnp.float32),
    )(*targs)
